# baseline (speedup 1.0000x reference)
import functools

import jax
import jax.numpy as jnp
import numpy as np
from jax import lax
from jax.experimental import pallas as pl
from jax.experimental.pallas import tpu as pltpu
from jax.experimental.pallas import tpu_sc as plsc

N_ROWS = 100000
D = 128
S = 512
NC, NS, L = 2, 16, 16
NW = NC * NS
CHUNK = 80
NCHUNKS = N_ROWS // CHUNK
MAXC = -(-NCHUNKS // NW)
KPER = D // L
NGRP = CHUNK // L


def _sc_partial(x2, ids):
  mesh = plsc.VectorSubcoreMesh(core_axis_name="c", subcore_axis_name="s")

  @functools.partial(
      pl.kernel,
      out_type=[
          jax.ShapeDtypeStruct((NC, S, D), jnp.float32),
          jax.ShapeDtypeStruct((NC, S), jnp.float32),
      ],
      mesh=mesh,
      compiler_params=pltpu.CompilerParams(needs_layout_passes=False),
      scratch_types=[
          pltpu.VMEM((CHUNK, D), jnp.float32),
          pltpu.VMEM((CHUNK, D), jnp.float32),
          pltpu.VMEM((CHUNK, D), jnp.float32),
          pltpu.VMEM((CHUNK, D), jnp.float32),
          pltpu.VMEM((CHUNK,), jnp.int32),
          pltpu.VMEM((CHUNK,), jnp.int32),
          pltpu.VMEM((CHUNK,), jnp.int32),
          pltpu.VMEM((CHUNK,), jnp.int32),
          pltpu.VMEM((S, D), jnp.float32),
          pltpu.VMEM((S,), jnp.float32),
          pltpu.VMEM((S,), jnp.int32),
          pltpu.VMEM_SHARED((S, D), jnp.float32),
          pltpu.VMEM_SHARED((S,), jnp.float32),
          pltpu.SemaphoreType.DMA,
          pltpu.SemaphoreType.DMA,
          pltpu.SemaphoreType.DMA,
          pltpu.SemaphoreType.DMA,
          pltpu.SemaphoreType.DMA,
          pltpu.SemaphoreType.DMA,
          pltpu.SemaphoreType.DMA,
          pltpu.SemaphoreType.DMA,
      ],
  )
  def k(x_hbm, ids_hbm, psum_hbm, pcnt_hbm, xv0, xv1, xv2, xv3,
        iv0, iv1, iv2, iv3, acc, cnt, rowidx, shacc, shcnt,
        sx0, sx1, sx2, sx3, si0, si1, si2, si3):
    wid = lax.axis_index("s") * NC + lax.axis_index("c")
    xv = (xv0, xv1, xv2, xv3)
    iv = (iv0, iv1, iv2, iv3)
    sx = (sx0, sx1, sx2, sx3)
    si = (si0, si1, si2, si3)

    def start(slot, i):
      valid = wid + NW * i < NCHUNKS

      @pl.when(valid)
      def _():
        row0 = (wid + NW * i) * CHUNK
        pltpu.async_copy(x_hbm.at[pl.ds(row0, CHUNK), :], xv[slot], sx[slot])
        pltpu.async_copy(ids_hbm.at[pl.ds(row0, CHUNK)], iv[slot], si[slot])

    def wait(slot, i):
      valid = wid + NW * i < NCHUNKS

      @pl.when(valid)
      def _():
        pltpu.make_async_copy(x_hbm.at[pl.ds(0, CHUNK), :], xv[slot],
                              sx[slot]).wait()
        pltpu.make_async_copy(ids_hbm.at[pl.ds(0, CHUNK)], iv[slot],
                              si[slot]).wait()

    start(0, 0)

    zf = jnp.zeros((L,), jnp.float32)
    bidx = jnp.arange(L, dtype=jnp.int32)

    def zbody(j, _):
      for u in range(2):
        for kk in range(KPER):
          acc[2 * j + u, pl.ds(kk * L, L)] = zf
      return 0

    lax.fori_loop(0, S // 2, zbody, 0)

    def zcnt(j, _):
      cnt[pl.ds(j * L, L)] = zf
      rowidx[pl.ds(j * L, L)] = bidx + j * L
      return 0

    lax.fori_loop(0, S // L, zcnt, 0)

    cols = [jnp.arange(kk * L, (kk + 1) * L, dtype=jnp.int32)
            for kk in range(KPER)]
    ones = jnp.ones((L,), jnp.float32)
    lane0 = jnp.arange(L, dtype=jnp.int32) == 0
    lanes = [jnp.full((L,), u, dtype=jnp.int32) for u in range(L)]
    shift1 = jnp.maximum(jnp.arange(L, dtype=jnp.int32) - 1, 0)
    zacc = tuple(jnp.zeros((L,), jnp.float32) for _ in range(KPER))

    def load_row(xvs, r):
      return [xvs[r, pl.ds(kk * L, L)] for kk in range(KPER)]

    def scatter_row(idu, xs):
      for kk in range(KPER):
        plsc.addupdate_scatter(acc, [idu, cols[kk]], xs[kk])
      plsc.addupdate_scatter(cnt, [idu], ones, mask=lane0)

    def flush(c_id, c_acc, c_cnt):
      for kk in range(KPER):
        plsc.addupdate_scatter(acc, [c_id, cols[kk]], c_acc[kk])
      plsc.addupdate_scatter(cnt, [c_id], c_cnt, mask=lane0)

    def process(slot, i):
      valid = wid + NW * i < NCHUNKS
      xvs, ivs = xv[slot], iv[slot]

      @pl.when(valid)
      def _():
        idvec0 = ivs[pl.ds(0, L)]
        init = (jnp.take_along_axis(idvec0, lanes[0], 0), zacc,
                jnp.zeros((L,), jnp.float32))

        def grp(j, carry):
          c_id, c_acc, c_cnt = carry
          idvec = ivs[pl.ds(j * L, L)]
          prev = jnp.take_along_axis(idvec, shift1, 0)
          prev = jnp.where(lane0, c_id, prev)
          no_boundary = jnp.all(idvec == prev)

          def fast(c):
            c_id, c_acc, c_cnt = c
            new_acc = []
            for kk in range(KPER):
              vals = [xvs[j * L + u, pl.ds(kk * L, L)] for u in range(L)]
              while len(vals) > 1:
                vals = [vals[m] + vals[m + 1] for m in range(0, len(vals), 2)]
              new_acc.append(c_acc[kk] + vals[0])
            return (c_id, tuple(new_acc), c_cnt + np.float32(L))

          def slow(c):
            c_id, c_acc, c_cnt = c
            flush(c_id, c_acc, c_cnt)
            for u in range(L):
              idu = jnp.take_along_axis(idvec, lanes[u], 0)
              scatter_row(idu, load_row(xvs, j * L + u))
            return (jnp.take_along_axis(idvec, lanes[L - 1], 0), zacc,
                    jnp.zeros((L,), jnp.float32))

          return lax.cond(no_boundary, fast, slow, carry)

        c_id, c_acc, c_cnt = lax.fori_loop(0, NGRP, grp, init)
        flush(c_id, c_acc, c_cnt)

    start(1, 1)
    start(2, 2)
    start(3, 3)

    def outer(t, _):
      i0 = 4 * t
      for p in range(4):
        wait(p, i0 + p)
        process(p, i0 + p)
        start(p, i0 + p + 4)
      return 0

    lax.fori_loop(0, MAXC // 4, outer, 0)

    sid = lax.axis_index("s")
    cid = lax.axis_index("c")

    @pl.when(sid == 0)
    def _():
      pltpu.sync_copy(acc, shacc)
      pltpu.sync_copy(cnt, shcnt)

    plsc.subcore_barrier()

    @pl.when(sid != 0)
    def _():
      pltpu.sync_copy(acc, shacc.at[rowidx], add=True)
      pltpu.sync_copy(cnt, shcnt.at[rowidx], add=True)

    plsc.subcore_barrier()

    @pl.when(sid == 0)
    def _():
      pltpu.sync_copy(shacc, psum_hbm.at[cid])
      pltpu.sync_copy(shcnt, pcnt_hbm.at[cid])

  return k(x2, ids)


def _tc_reduce(psum, pcnt):
  BS = 512

  def body(ps_ref, pc_ref, o_ref):
    s = jnp.sum(ps_ref[...], axis=0)
    c = jnp.sum(pc_ref[...], axis=0)
    m = s / jnp.clip(c, 1.0, None)[:, None]
    o_ref[...] = jnp.concatenate([s, m], axis=-1)

  return pl.pallas_call(
      body,
      grid=(S // BS,),
      in_specs=[
          pl.BlockSpec((NC, BS, D), lambda i: (0, i, 0)),
          pl.BlockSpec((NC, BS), lambda i: (0, i)),
      ],
      out_specs=pl.BlockSpec((BS, 2 * D), lambda i: (i, 0)),
      out_shape=jax.ShapeDtypeStruct((S, 2 * D), jnp.float32),
  )(psum, pcnt)


def kernel(x, batch):
  ids = batch.astype(jnp.int32)
  psum, pcnt = _sc_partial(x, ids)
  return _tc_reduce(psum, pcnt)


# speedup vs baseline: 1.3032x; 1.3032x over previous
import functools

import jax
import jax.numpy as jnp
import numpy as np
from jax import lax
from jax.experimental import pallas as pl
from jax.experimental.pallas import tpu as pltpu
from jax.experimental.pallas import tpu_sc as plsc

N_ROWS = 100000
D = 128
S = 512
NC, NS, L = 2, 16, 16
NW = NC * NS
CHUNK = 200
NCHUNKS = N_ROWS // CHUNK
MAXC = -(-NCHUNKS // NW)
KPER = D // L
NGRP = CHUNK // L
TAIL = CHUNK - NGRP * L


def _sc_partial(x2, ids):
  mesh = plsc.VectorSubcoreMesh(core_axis_name="c", subcore_axis_name="s")

  @functools.partial(
      pl.kernel,
      out_type=[
          jax.ShapeDtypeStruct((NC, S, D), jnp.float32),
          jax.ShapeDtypeStruct((NC, S), jnp.float32),
      ],
      mesh=mesh,
      compiler_params=pltpu.CompilerParams(needs_layout_passes=False),
      scratch_types=[
          pltpu.VMEM((CHUNK, D), jnp.float32),
          pltpu.VMEM((CHUNK, D), jnp.float32),
          pltpu.VMEM((CHUNK,), jnp.int32),
          pltpu.VMEM((CHUNK,), jnp.int32),
          pltpu.VMEM((S, D), jnp.float32),
          pltpu.VMEM((S,), jnp.float32),
          pltpu.VMEM((S,), jnp.int32),
          pltpu.VMEM_SHARED((S, D), jnp.float32),
          pltpu.VMEM_SHARED((S,), jnp.float32),
          pltpu.SemaphoreType.DMA,
          pltpu.SemaphoreType.DMA,
          pltpu.SemaphoreType.DMA,
          pltpu.SemaphoreType.DMA,
      ],
  )
  def k(x_hbm, ids_hbm, psum_hbm, pcnt_hbm, xv0, xv1, iv0, iv1, acc, cnt,
        rowidx, shacc, shcnt, sx0, sx1, si0, si1):
    wid = lax.axis_index("s") * NC + lax.axis_index("c")
    xv = (xv0, xv1)
    iv = (iv0, iv1)
    sx = (sx0, sx1)
    si = (si0, si1)

    def start(slot, i):
      valid = wid + NW * i < NCHUNKS

      @pl.when(valid)
      def _():
        row0 = (wid + NW * i) * CHUNK
        pltpu.async_copy(x_hbm.at[pl.ds(row0, CHUNK), :], xv[slot], sx[slot])
        pltpu.async_copy(ids_hbm.at[pl.ds(row0, CHUNK)], iv[slot], si[slot])

    def wait(slot, i):
      valid = wid + NW * i < NCHUNKS

      @pl.when(valid)
      def _():
        pltpu.make_async_copy(x_hbm.at[pl.ds(0, CHUNK), :], xv[slot],
                              sx[slot]).wait()
        pltpu.make_async_copy(ids_hbm.at[pl.ds(0, CHUNK)], iv[slot],
                              si[slot]).wait()

    start(0, 0)

    zf = jnp.zeros((L,), jnp.float32)
    bidx = jnp.arange(L, dtype=jnp.int32)

    def zbody(j, _):
      for u in range(2):
        for kk in range(KPER):
          acc[2 * j + u, pl.ds(kk * L, L)] = zf
      return 0

    lax.fori_loop(0, S // 2, zbody, 0)

    def zcnt(j, _):
      cnt[pl.ds(j * L, L)] = zf
      rowidx[pl.ds(j * L, L)] = bidx + j * L
      return 0

    lax.fori_loop(0, S // L, zcnt, 0)

    cols = [jnp.arange(kk * L, (kk + 1) * L, dtype=jnp.int32)
            for kk in range(KPER)]
    ones = jnp.ones((L,), jnp.float32)
    lane0 = jnp.arange(L, dtype=jnp.int32) == 0
    lanes = [jnp.full((L,), u, dtype=jnp.int32) for u in range(L)]
    shift1 = jnp.maximum(jnp.arange(L, dtype=jnp.int32) - 1, 0)
    zacc = tuple(jnp.zeros((L,), jnp.float32) for _ in range(KPER))

    def load_row(xvs, r):
      return [xvs[r, pl.ds(kk * L, L)] for kk in range(KPER)]

    def scatter_row(idu, xs):
      for kk in range(KPER):
        plsc.addupdate_scatter(acc, [idu, cols[kk]], xs[kk])
      plsc.addupdate_scatter(cnt, [idu], ones, mask=lane0)

    def flush(c_id, c_acc, c_cnt):
      for kk in range(KPER):
        plsc.addupdate_scatter(acc, [c_id, cols[kk]], c_acc[kk])
      plsc.addupdate_scatter(cnt, [c_id], c_cnt, mask=lane0)

    def process(slot, i):
      valid = wid + NW * i < NCHUNKS
      xvs, ivs = xv[slot], iv[slot]

      @pl.when(valid)
      def _():
        idvec0 = ivs[pl.ds(0, L)]
        init = (jnp.take_along_axis(idvec0, lanes[0], 0), zacc,
                jnp.zeros((L,), jnp.float32))

        def grp(j, carry):
          c_id, c_acc, c_cnt = carry
          idvec = ivs[pl.ds(j * L, L)]
          prev = jnp.take_along_axis(idvec, shift1, 0)
          prev = jnp.where(lane0, c_id, prev)
          no_boundary = jnp.all(idvec == prev)

          def fast(c):
            c_id, c_acc, c_cnt = c
            new_acc = []
            for kk in range(KPER):
              vals = [xvs[j * L + u, pl.ds(kk * L, L)] for u in range(L)]
              while len(vals) > 1:
                vals = [vals[m] + vals[m + 1] for m in range(0, len(vals), 2)]
              new_acc.append(c_acc[kk] + vals[0])
            return (c_id, tuple(new_acc), c_cnt + np.float32(L))

          def slow(c):
            c_id, c_acc, c_cnt = c
            flush(c_id, c_acc, c_cnt)
            for u in range(L):
              idu = jnp.take_along_axis(idvec, lanes[u], 0)
              scatter_row(idu, load_row(xvs, j * L + u))
            return (jnp.take_along_axis(idvec, lanes[L - 1], 0), zacc,
                    jnp.zeros((L,), jnp.float32))

          return lax.cond(no_boundary, fast, slow, carry)

        c_id, c_acc, c_cnt = lax.fori_loop(0, NGRP, grp, init)
        flush(c_id, c_acc, c_cnt)
        idvec = ivs[pl.ds(CHUNK - L, L)]
        for u in range(L - TAIL, L):
          idu = jnp.take_along_axis(idvec, lanes[u], 0)
          scatter_row(idu, load_row(xvs, CHUNK - L + u))

    start(1, 1)

    def outer(t, _):
      i0 = 2 * t
      wait(0, i0)
      process(0, i0)
      start(0, i0 + 2)
      wait(1, i0 + 1)
      process(1, i0 + 1)
      start(1, i0 + 3)
      return 0

    lax.fori_loop(0, MAXC // 2, outer, 0)

    sid = lax.axis_index("s")
    cid = lax.axis_index("c")

    @pl.when(sid == 0)
    def _():
      pltpu.sync_copy(acc, shacc)
      pltpu.sync_copy(cnt, shcnt)

    plsc.subcore_barrier()

    @pl.when(sid != 0)
    def _():
      pltpu.sync_copy(acc, shacc.at[rowidx], add=True)
      pltpu.sync_copy(cnt, shcnt.at[rowidx], add=True)

    plsc.subcore_barrier()

    @pl.when(sid == 0)
    def _():
      pltpu.sync_copy(shacc, psum_hbm.at[cid])
      pltpu.sync_copy(shcnt, pcnt_hbm.at[cid])

  return k(x2, ids)


def _tc_reduce(psum, pcnt):
  BS = 512

  def body(ps_ref, pc_ref, o_ref):
    s = jnp.sum(ps_ref[...], axis=0)
    c = jnp.sum(pc_ref[...], axis=0)
    m = s / jnp.clip(c, 1.0, None)[:, None]
    o_ref[...] = jnp.concatenate([s, m], axis=-1)

  return pl.pallas_call(
      body,
      grid=(S // BS,),
      in_specs=[
          pl.BlockSpec((NC, BS, D), lambda i: (0, i, 0)),
          pl.BlockSpec((NC, BS), lambda i: (0, i)),
      ],
      out_specs=pl.BlockSpec((BS, 2 * D), lambda i: (i, 0)),
      out_shape=jax.ShapeDtypeStruct((S, 2 * D), jnp.float32),
  )(psum, pcnt)


def kernel(x, batch):
  ids = batch.astype(jnp.int32)
  psum, pcnt = _sc_partial(x, ids)
  return _tc_reduce(psum, pcnt)
